# Initial kernel scaffold; baseline (speedup 1.0000x reference)
import jax
import jax.numpy as jnp
from jax.experimental import pallas as pl


def kernel(xyz, points, enc_w, enc_b, enc_g, enc_beta, b0_ew, b0_eb, b0_lg, b0_lb, b0_pw, b0_pb, b0_sw, b1_ew, b1_eb, b1_lg, b1_lb, b1_pw, b1_pb, b1_sw):
    raise NotImplementedError("write your pallas kernel here")


# speedup vs baseline: 6.8908x; 6.8908x over previous
import functools

import numpy as np
import jax
import jax.numpy as jnp
from jax import lax
from jax.experimental import pallas as pl
from jax.experimental.pallas import tpu as pltpu
from jax.experimental.pallas import tpu_sc as plsc

B = 2
N = 16384
NPOINT = 1024
NSAMPLE = 32
RADIUS2 = np.float32(0.1 ** 2)
IN_CH = 32
TW = 128
NTILES = 32
CPT = (B * NPOINT) // NTILES
GCH = 128
EPS = 1e-5
SB = 8


def _fps_body(xyz_ref, out_ref):
    x = xyz_ref[0, 0]
    y = xyz_ref[0, 1]
    z = xyz_ref[0, 2]
    row = lax.broadcasted_iota(jnp.int32, (128, 128), 0)
    col = lax.broadcasted_iota(jnp.int32, (128, 128), 1)
    iota = row * 128 + col
    prow = lax.broadcasted_iota(jnp.int32, (8, 128), 0)
    pcol = lax.broadcasted_iota(jnp.int32, (8, 128), 1)
    piota = prow * 128 + pcol

    def body(i, state):
        far, dist, ax, ay, az = state
        eq = iota == far
        cx = jnp.sum(jnp.where(eq, x, 0.0))
        cy = jnp.sum(jnp.where(eq, y, 0.0))
        cz = jnp.sum(jnp.where(eq, z, 0.0))
        sel = piota == i
        ax = jnp.where(sel, cx, ax)
        ay = jnp.where(sel, cy, ay)
        az = jnp.where(sel, cz, az)
        dx = x - cx
        dy = y - cy
        dz = z - cz
        d = (dx * dx + dy * dy) + dz * dz
        dist = jnp.minimum(dist, d)
        m = jnp.max(dist)
        far = jnp.min(jnp.where(dist == m, iota, N))
        return far, dist, ax, ay, az

    init = (jnp.int32(0), jnp.full((128, 128), 1e10, jnp.float32),
            jnp.zeros((8, 128), jnp.float32), jnp.zeros((8, 128), jnp.float32),
            jnp.zeros((8, 128), jnp.float32))
    _, _, ax, ay, az = lax.fori_loop(0, NPOINT, body, init)
    out_ref[0, 0] = ax
    out_ref[0, 1] = ay
    out_ref[0, 2] = az


def _fps(xyz4):
    return pl.pallas_call(
        _fps_body,
        grid=(B,),
        in_specs=[pl.BlockSpec((1, 3, 128, 128), lambda b: (b, 0, 0, 0))],
        out_specs=pl.BlockSpec((1, 3, 8, 128), lambda b: (b, 0, 0, 0)),
        out_shape=jax.ShapeDtypeStruct((B, 3, 8, 128), jnp.float32),
    )(xyz4)


def _bq_body(xyz_ref, xyzr_ref, c_ref, cr_ref, out_ref):
    b = pl.program_id(0)
    xs = xyz_ref[0]
    s2 = (xs[0:1] * xs[0:1] + xs[1:2] * xs[1:2]) + xs[2:3] * xs[2:3]
    c = c_ref[...]
    s1 = (c[:, 0:1] * c[:, 0:1] + c[:, 1:2] * c[:, 1:2]) + c[:, 2:3] * c[:, 2:3]
    dot = jnp.dot(cr_ref[...].astype(jnp.bfloat16),
                  xyzr_ref[0].astype(jnp.bfloat16),
                  preferred_element_type=jnp.float32)
    sqr = (s1 + s2) - 2.0 * dot
    mask = sqr <= RADIUS2
    iota_n = lax.broadcasted_iota(jnp.int32, (SB, N), 1)
    val = jnp.where(mask, iota_n, N)
    jiota = lax.broadcasted_iota(jnp.int32, (SB, NSAMPLE), 1)
    idx = jnp.zeros((SB, NSAMPLE), jnp.int32)

    def ext(j, state):
        val, idx = state
        m = jnp.min(val, axis=1, keepdims=True)
        idx = jnp.where(jiota == j, m, idx)
        val = jnp.where(val == m, jnp.int32(2 ** 30), val)
        return val, idx

    _, idx = lax.fori_loop(0, NSAMPLE, ext, (val, idx))
    first = idx[:, 0:1]
    idx = jnp.where(idx >= N, first, idx)
    idx = jnp.where(idx >= N, N - 1, idx)
    out_ref[...] = idx + b * N


def _ball_query(xyz_t, xyz_rnd_t, cent, cent_rnd):
    return pl.pallas_call(
        _bq_body,
        grid=(B, NPOINT // SB),
        in_specs=[
            pl.BlockSpec((1, 3, N), lambda b, i: (b, 0, 0)),
            pl.BlockSpec((1, 3, N), lambda b, i: (b, 0, 0)),
            pl.BlockSpec((SB, 3), lambda b, i: (b * (NPOINT // SB) + i, 0)),
            pl.BlockSpec((SB, 3), lambda b, i: (b * (NPOINT // SB) + i, 0)),
        ],
        out_specs=pl.BlockSpec((SB, NSAMPLE),
                               lambda b, i: (b * (NPOINT // SB) + i, 0)),
        out_shape=jax.ShapeDtypeStruct((B * NPOINT, NSAMPLE), jnp.int32),
    )(xyz_t, xyz_rnd_t, cent, cent_rnd)


def _sc_gather_body(idx_hbm, table, out, idx_v, rows, sem):
    wid = lax.axis_index("s") * 2 + lax.axis_index("c")
    base = wid * (CPT * NSAMPLE)
    pltpu.sync_copy(idx_hbm.at[pl.ds(base, CPT * NSAMPLE)], idx_v)

    def chunk(t, _):
        o = t * GCH
        pltpu.async_copy(table.at[idx_v.at[pl.ds(o, GCH)]], rows, sem).wait()
        pltpu.sync_copy(rows, out.at[pl.ds(base + o, GCH)])
        return 0

    lax.fori_loop(0, (CPT * NSAMPLE) // GCH, chunk, 0)


def _sc_gather(idx_flat, table):
    mesh = plsc.VectorSubcoreMesh(core_axis_name="c", subcore_axis_name="s")
    fn = functools.partial(
        pl.kernel,
        mesh=mesh,
        out_type=jax.ShapeDtypeStruct((B * NPOINT * NSAMPLE, TW), jnp.float32),
        scratch_types=[
            pltpu.VMEM((CPT * NSAMPLE,), jnp.int32),
            pltpu.VMEM((GCH, TW), jnp.float32),
            pltpu.SemaphoreType.DMA,
        ],
    )(_sc_gather_body)
    return fn(idx_flat, table)


def _bf16_dot(a, w):
    return jnp.dot(a.astype(jnp.bfloat16), w.astype(jnp.bfloat16),
                   preferred_element_type=jnp.float32)


def _layernorm(x, g, b):
    mu = jnp.mean(x, axis=-1, keepdims=True)
    var = jnp.mean((x - mu) ** 2, axis=-1, keepdims=True)
    return (x - mu) / jnp.sqrt(var + EPS) * g + b


def _gelu(x):
    return 0.5 * x * (1.0 + lax.erf(x * np.float32(np.sqrt(0.5))))


CB = 64
RB = CB * NSAMPLE


def _mlp_body(g_ref, nx_ref, ew_ref, ebias_ref, eg_ref, ebeta_ref,
              a_ew, a_eb, a_lg, a_lb, a_pw, a_pb, a_sw,
              c_ew, c_eb, c_lg, c_lb, c_pw, c_pb, c_sw, out_ref):
    g = g_ref[...]
    pts = g[:, :IN_CH]
    gx = g[:, IN_CH:IN_CH + 3]
    nx = nx_ref[...]
    nxb = jnp.broadcast_to(nx[:, None, :], (CB, NSAMPLE, 3)).reshape(RB, 3)
    rel = gx - nxb
    h = _bf16_dot(rel, ew_ref[...]) + ebias_ref[...]
    feat = _gelu(_layernorm(h, eg_ref[...], ebeta_ref[...]))
    comb = jnp.concatenate([pts, feat], axis=-1)

    def irb(x, ew, eb, lg, lb, pw, pb, sw):
        hh = _gelu(_layernorm(_bf16_dot(x, ew[...]) + eb[...], lg[...], lb[...]))
        return _bf16_dot(hh, pw[...]) + pb[...] + _bf16_dot(x, sw[...])

    o = irb(comb, a_ew, a_eb, a_lg, a_lb, a_pw, a_pb, a_sw)
    o = irb(o, c_ew, c_eb, c_lg, c_lb, c_pw, c_pb, c_sw)
    o = o.reshape(CB, NSAMPLE, 64)
    out_ref[...] = jnp.max(o, axis=1)


def _mlp(gathered, cent, enc_w, enc_b, enc_g, enc_beta,
         b0_ew, b0_eb, b0_lg, b0_lb, b0_pw, b0_pb, b0_sw,
         b1_ew, b1_eb, b1_lg, b1_lb, b1_pw, b1_pb, b1_sw):
    nsteps = (B * NPOINT) // CB
    full = lambda shape: pl.BlockSpec(shape, lambda i: (0,) * len(shape))
    row2 = lambda m: full((1, m))
    args = [gathered, cent,
            enc_w, enc_b.reshape(1, -1), enc_g.reshape(1, -1),
            enc_beta.reshape(1, -1),
            b0_ew, b0_eb.reshape(1, -1), b0_lg.reshape(1, -1),
            b0_lb.reshape(1, -1), b0_pw, b0_pb.reshape(1, -1), b0_sw,
            b1_ew, b1_eb.reshape(1, -1), b1_lg.reshape(1, -1),
            b1_lb.reshape(1, -1), b1_pw, b1_pb.reshape(1, -1), b1_sw]
    specs = [pl.BlockSpec((RB, TW), lambda i: (i, 0)),
             pl.BlockSpec((CB, 3), lambda i: (i, 0)),
             full((3, IN_CH)), row2(IN_CH), row2(IN_CH), row2(IN_CH),
             full((64, 256)), row2(256), row2(256), row2(256),
             full((256, 32)), row2(32), full((64, 32)),
             full((32, 128)), row2(128), row2(128), row2(128),
             full((128, 64)), row2(64), full((32, 64))]
    return pl.pallas_call(
        _mlp_body,
        grid=(nsteps,),
        in_specs=specs,
        out_specs=pl.BlockSpec((CB, 64), lambda i: (i, 0)),
        out_shape=jax.ShapeDtypeStruct((B * NPOINT, 64), jnp.float32),
    )(*args)


def kernel(xyz, points, enc_w, enc_b, enc_g, enc_beta,
           b0_ew, b0_eb, b0_lg, b0_lb, b0_pw, b0_pb, b0_sw,
           b1_ew, b1_eb, b1_lg, b1_lb, b1_pw, b1_pb, b1_sw):
    xyz_t = jnp.transpose(xyz, (0, 2, 1))
    xyz4 = xyz_t.reshape(B, 3, 128, 128)
    nxc = _fps(xyz4)
    new_xyz = jnp.transpose(nxc.reshape(B, 3, NPOINT), (0, 2, 1))

    xyz_rnd_t = xyz_t.astype(jnp.bfloat16).astype(jnp.float32)
    cent = new_xyz.reshape(B * NPOINT, 3)
    cent_rnd = cent.astype(jnp.bfloat16).astype(jnp.float32)

    idx = _ball_query(xyz_t, xyz_rnd_t, cent, cent_rnd)
    idx_flat = idx.reshape(B * NPOINT * NSAMPLE)

    table = jnp.concatenate(
        [points, xyz, jnp.zeros((B, N, TW - IN_CH - 3), jnp.float32)],
        axis=-1).reshape(B * N, TW)
    gathered = _sc_gather(idx_flat, table)

    out = _mlp(gathered, cent, enc_w, enc_b, enc_g, enc_beta,
               b0_ew, b0_eb, b0_lg, b0_lb, b0_pw, b0_pb, b0_sw,
               b1_ew, b1_eb, b1_lg, b1_lb, b1_pw, b1_pb, b1_sw)
    new_points = out.reshape(B, NPOINT, 64)
    return (new_xyz, new_points)


# speedup vs baseline: 6.9221x; 1.0045x over previous
import functools

import numpy as np
import jax
import jax.numpy as jnp
from jax import lax
from jax.experimental import pallas as pl
from jax.experimental.pallas import tpu as pltpu
from jax.experimental.pallas import tpu_sc as plsc

B = 2
N = 16384
NPOINT = 1024
NSAMPLE = 32
RADIUS2 = np.float32(0.1 ** 2)
IN_CH = 32
TW = 128
NTILES = 32
CPT = (B * NPOINT) // NTILES
GCH = 128
EPS = 1e-5
SB = 8


def _fps_body(xyz_ref, out_ref):
    x = xyz_ref[0, 0]
    y = xyz_ref[0, 1]
    z = xyz_ref[0, 2]
    row = lax.broadcasted_iota(jnp.int32, (128, 128), 0)
    col = lax.broadcasted_iota(jnp.int32, (128, 128), 1)
    iota = row * 128 + col
    prow = lax.broadcasted_iota(jnp.int32, (8, 128), 0)
    pcol = lax.broadcasted_iota(jnp.int32, (8, 128), 1)
    piota = prow * 128 + pcol

    def body(i, state):
        far, dist, ax, ay, az = state
        eq = iota == far
        cx = jnp.sum(jnp.where(eq, x, 0.0))
        cy = jnp.sum(jnp.where(eq, y, 0.0))
        cz = jnp.sum(jnp.where(eq, z, 0.0))
        sel = piota == i
        ax = jnp.where(sel, cx, ax)
        ay = jnp.where(sel, cy, ay)
        az = jnp.where(sel, cz, az)
        dx = x - cx
        dy = y - cy
        dz = z - cz
        d = (dx * dx + dy * dy) + dz * dz
        dist = jnp.minimum(dist, d)
        m = jnp.max(dist)
        far = jnp.min(jnp.where(dist == m, iota, N))
        return far, dist, ax, ay, az

    init = (jnp.int32(0), jnp.full((128, 128), 1e10, jnp.float32),
            jnp.zeros((8, 128), jnp.float32), jnp.zeros((8, 128), jnp.float32),
            jnp.zeros((8, 128), jnp.float32))
    _, _, ax, ay, az = lax.fori_loop(0, NPOINT, body, init)
    out_ref[0, 0] = ax
    out_ref[0, 1] = ay
    out_ref[0, 2] = az


def _fps(xyz4):
    return pl.pallas_call(
        _fps_body,
        grid=(B,),
        in_specs=[pl.BlockSpec((1, 3, 128, 128), lambda b: (b, 0, 0, 0))],
        out_specs=pl.BlockSpec((1, 3, 8, 128), lambda b: (b, 0, 0, 0)),
        out_shape=jax.ShapeDtypeStruct((B, 3, 8, 128), jnp.float32),
    )(xyz4)


def _bq_body(xyz_ref, xyzr_ref, c_ref, cr_ref, out_ref):
    b = pl.program_id(0)
    xs = xyz_ref[0]
    s2 = (xs[0:1] * xs[0:1] + xs[1:2] * xs[1:2]) + xs[2:3] * xs[2:3]
    c = c_ref[...]
    s1 = (c[:, 0:1] * c[:, 0:1] + c[:, 1:2] * c[:, 1:2]) + c[:, 2:3] * c[:, 2:3]
    dot = jnp.dot(cr_ref[...].astype(jnp.bfloat16),
                  xyzr_ref[0].astype(jnp.bfloat16),
                  preferred_element_type=jnp.float32)
    sqr = (s1 + s2) - 2.0 * dot
    mask = sqr <= RADIUS2
    iota_n = lax.broadcasted_iota(jnp.int32, (SB, N), 1)
    val = jnp.where(mask, iota_n, N)
    jiota = lax.broadcasted_iota(jnp.int32, (SB, NSAMPLE), 1)

    idx = jnp.zeros((SB, NSAMPLE), jnp.int32)

    def ext(j, state):
        v, idx = state
        m = jnp.min(v, axis=1, keepdims=True)
        idx = jnp.where(jiota == j, m, idx)
        v = jnp.where(v == m, jnp.int32(2 ** 30), v)
        return v, idx

    _, idx = lax.fori_loop(0, NSAMPLE, ext, (val, idx))
    first = idx[:, 0:1]
    idx = jnp.where(idx >= N, first, idx)
    idx = jnp.where(idx >= N, N - 1, idx)
    out_ref[...] = idx + b * N


def _ball_query(xyz_t, xyz_rnd_t, cent, cent_rnd):
    return pl.pallas_call(
        _bq_body,
        grid=(B, NPOINT // SB),
        in_specs=[
            pl.BlockSpec((1, 3, N), lambda b, i: (b, 0, 0)),
            pl.BlockSpec((1, 3, N), lambda b, i: (b, 0, 0)),
            pl.BlockSpec((SB, 3), lambda b, i: (b * (NPOINT // SB) + i, 0)),
            pl.BlockSpec((SB, 3), lambda b, i: (b * (NPOINT // SB) + i, 0)),
        ],
        out_specs=pl.BlockSpec((SB, NSAMPLE),
                               lambda b, i: (b * (NPOINT // SB) + i, 0)),
        out_shape=jax.ShapeDtypeStruct((B * NPOINT, NSAMPLE), jnp.int32),
    )(xyz_t, xyz_rnd_t, cent, cent_rnd)


def _sc_gather_body(idx_hbm, table, out, idx_v, rows, sem):
    wid = lax.axis_index("s") * 2 + lax.axis_index("c")
    base = wid * (CPT * NSAMPLE)
    pltpu.sync_copy(idx_hbm.at[pl.ds(base, CPT * NSAMPLE)], idx_v)

    def chunk(t, _):
        o = t * GCH
        pltpu.async_copy(table.at[idx_v.at[pl.ds(o, GCH)]], rows, sem).wait()
        pltpu.sync_copy(rows, out.at[pl.ds(base + o, GCH)])
        return 0

    lax.fori_loop(0, (CPT * NSAMPLE) // GCH, chunk, 0)


def _sc_gather(idx_flat, table):
    mesh = plsc.VectorSubcoreMesh(core_axis_name="c", subcore_axis_name="s")
    fn = functools.partial(
        pl.kernel,
        mesh=mesh,
        out_type=jax.ShapeDtypeStruct((B * NPOINT * NSAMPLE, TW), jnp.float32),
        scratch_types=[
            pltpu.VMEM((CPT * NSAMPLE,), jnp.int32),
            pltpu.VMEM((GCH, TW), jnp.float32),
            pltpu.SemaphoreType.DMA,
        ],
    )(_sc_gather_body)
    return fn(idx_flat, table)


def _bf16_dot(a, w):
    return jnp.dot(a.astype(jnp.bfloat16), w.astype(jnp.bfloat16),
                   preferred_element_type=jnp.float32)


def _layernorm(x, g, b):
    mu = jnp.mean(x, axis=-1, keepdims=True)
    var = jnp.mean((x - mu) ** 2, axis=-1, keepdims=True)
    return (x - mu) / jnp.sqrt(var + EPS) * g + b


def _gelu(x):
    return 0.5 * x * (1.0 + lax.erf(x * np.float32(np.sqrt(0.5))))


CB = 64
RB = CB * NSAMPLE


def _mlp_body(g_ref, nx_ref, ew_ref, ebias_ref, eg_ref, ebeta_ref,
              a_ew, a_eb, a_lg, a_lb, a_pw, a_pb, a_sw,
              c_ew, c_eb, c_lg, c_lb, c_pw, c_pb, c_sw, out_ref):
    g = g_ref[...]
    pts = g[:, :IN_CH]
    gx = g[:, IN_CH:IN_CH + 3]
    nx = nx_ref[...]
    nxb = jnp.broadcast_to(nx[:, None, :], (CB, NSAMPLE, 3)).reshape(RB, 3)
    rel = gx - nxb
    h = _bf16_dot(rel, ew_ref[...]) + ebias_ref[...]
    feat = _gelu(_layernorm(h, eg_ref[...], ebeta_ref[...]))
    comb = jnp.concatenate([pts, feat], axis=-1)

    def irb(x, ew, eb, lg, lb, pw, pb, sw):
        hh = _gelu(_layernorm(_bf16_dot(x, ew[...]) + eb[...], lg[...], lb[...]))
        return _bf16_dot(hh, pw[...]) + pb[...] + _bf16_dot(x, sw[...])

    o = irb(comb, a_ew, a_eb, a_lg, a_lb, a_pw, a_pb, a_sw)
    o = irb(o, c_ew, c_eb, c_lg, c_lb, c_pw, c_pb, c_sw)
    o = o.reshape(CB, NSAMPLE, 64)
    out_ref[...] = jnp.max(o, axis=1)


def _mlp(gathered, cent, enc_w, enc_b, enc_g, enc_beta,
         b0_ew, b0_eb, b0_lg, b0_lb, b0_pw, b0_pb, b0_sw,
         b1_ew, b1_eb, b1_lg, b1_lb, b1_pw, b1_pb, b1_sw):
    nsteps = (B * NPOINT) // CB
    full = lambda shape: pl.BlockSpec(shape, lambda i: (0,) * len(shape))
    row2 = lambda m: full((1, m))
    args = [gathered, cent,
            enc_w, enc_b.reshape(1, -1), enc_g.reshape(1, -1),
            enc_beta.reshape(1, -1),
            b0_ew, b0_eb.reshape(1, -1), b0_lg.reshape(1, -1),
            b0_lb.reshape(1, -1), b0_pw, b0_pb.reshape(1, -1), b0_sw,
            b1_ew, b1_eb.reshape(1, -1), b1_lg.reshape(1, -1),
            b1_lb.reshape(1, -1), b1_pw, b1_pb.reshape(1, -1), b1_sw]
    specs = [pl.BlockSpec((RB, TW), lambda i: (i, 0)),
             pl.BlockSpec((CB, 3), lambda i: (i, 0)),
             full((3, IN_CH)), row2(IN_CH), row2(IN_CH), row2(IN_CH),
             full((64, 256)), row2(256), row2(256), row2(256),
             full((256, 32)), row2(32), full((64, 32)),
             full((32, 128)), row2(128), row2(128), row2(128),
             full((128, 64)), row2(64), full((32, 64))]
    return pl.pallas_call(
        _mlp_body,
        grid=(nsteps,),
        in_specs=specs,
        out_specs=pl.BlockSpec((CB, 64), lambda i: (i, 0)),
        out_shape=jax.ShapeDtypeStruct((B * NPOINT, 64), jnp.float32),
    )(*args)


def kernel(xyz, points, enc_w, enc_b, enc_g, enc_beta,
           b0_ew, b0_eb, b0_lg, b0_lb, b0_pw, b0_pb, b0_sw,
           b1_ew, b1_eb, b1_lg, b1_lb, b1_pw, b1_pb, b1_sw):
    xyz_t = jnp.transpose(xyz, (0, 2, 1))
    xyz4 = xyz_t.reshape(B, 3, 128, 128)
    nxc = _fps(xyz4)
    new_xyz = jnp.transpose(nxc.reshape(B, 3, NPOINT), (0, 2, 1))

    xyz_rnd_t = xyz_t.astype(jnp.bfloat16).astype(jnp.float32)
    cent = new_xyz.reshape(B * NPOINT, 3)
    cent_rnd = cent.astype(jnp.bfloat16).astype(jnp.float32)

    idx = _ball_query(xyz_t, xyz_rnd_t, cent, cent_rnd)
    idx_flat = idx.reshape(B * NPOINT * NSAMPLE)

    table = jnp.concatenate(
        [points, xyz, jnp.zeros((B, N, TW - IN_CH - 3), jnp.float32)],
        axis=-1).reshape(B * N, TW)
    gathered = _sc_gather(idx_flat, table)

    out = _mlp(gathered, cent, enc_w, enc_b, enc_g, enc_beta,
               b0_ew, b0_eb, b0_lg, b0_lb, b0_pw, b0_pb, b0_sw,
               b1_ew, b1_eb, b1_lg, b1_lb, b1_pw, b1_pb, b1_sw)
    new_points = out.reshape(B, NPOINT, 64)
    return (new_xyz, new_points)
